# Initial kernel scaffold; baseline (speedup 1.0000x reference)
import functools

import jax
import jax.numpy as jnp
from jax import lax
from jax.experimental import pallas as pl
from jax.experimental.pallas import tpu as pltpu
from jax.experimental.pallas import tpu_sc as plsc

N = 10000
E = 320000
D = 128
G = 64
DO = 10

NC = 2
NS = 16
NW = NC * NS
C = 128
CPW = 79
E_PAD = NW * CPW * C
RPW = N // NS

_mesh = plsc.VectorSubcoreMesh(core_axis_name="c", subcore_axis_name="s")


@functools.partial(
    pl.kernel,
    out_type=jax.ShapeDtypeStruct((NC, N, 16), jnp.float32),
    mesh=_mesh,
    scratch_types=[
        pltpu.VMEM((1, C), jnp.int32),
        pltpu.VMEM((C,), jnp.float32),
        pltpu.VMEM((C, 16), jnp.float32),
        pltpu.VMEM((RPW, 16), jnp.float32),
        pltpu.VMEM_SHARED((N, 16), jnp.float32),
        pltpu.SemaphoreType.DMA,
    ],
)
def _deg_kernel(dst_hbm, w_hbm, out_hbm, dst_v, w_v, val_v, zb_v, acc_sh, sem):
    del sem
    cid = lax.axis_index("c")
    sid = lax.axis_index("s")
    wid = cid * NS + sid
    zeros16 = jnp.zeros((16,), jnp.float32)

    @pl.loop(0, RPW)
    def _(r):
        zb_v[r, :] = zeros16

    @pl.loop(0, C)
    def _(r):
        val_v[r, :] = zeros16

    pltpu.sync_copy(zb_v, acc_sh.at[pl.ds(sid * RPW, RPW)])
    plsc.subcore_barrier()

    lanes = lax.iota(jnp.int32, 16)
    zlanes = jnp.zeros((16,), jnp.int32)

    @pl.loop(0, CPW)
    def _(c):
        base = (wid * CPW + c) * C
        pltpu.sync_copy(dst_hbm.at[pl.ds(base, C)], dst_v.at[0])
        pltpu.sync_copy(w_hbm.at[pl.ds(base, C)], w_v)
        for g in range(C // 16):
            vals = w_v[pl.ds(g * 16, 16)]
            plsc.store_scatter(val_v, [lanes + (g * 16), zlanes], vals)
        pltpu.sync_copy(val_v, acc_sh.at[dst_v.at[0]], add=True)

    plsc.subcore_barrier()
    pltpu.sync_copy(acc_sh.at[pl.ds(sid * RPW, RPW)],
                    out_hbm.at[cid].at[pl.ds(sid * RPW, RPW)])


@functools.partial(
    pl.kernel,
    out_type=jax.ShapeDtypeStruct((E_PAD,), jnp.float32),
    mesh=_mesh,
    scratch_types=[
        pltpu.VMEM((N,), jnp.float32),
        pltpu.VMEM((1, C), jnp.int32),
        pltpu.VMEM((1, C), jnp.int32),
        pltpu.VMEM((C,), jnp.float32),
        pltpu.VMEM((C,), jnp.float32),
    ],
)
def _norm_kernel(src_hbm, dst_hbm, w_hbm, dinv_hbm, out_hbm,
                 dinv_v, src_v, dst_v, w_v, nb_v):
    cid = lax.axis_index("c")
    sid = lax.axis_index("s")
    wid = cid * NS + sid
    pltpu.sync_copy(dinv_hbm, dinv_v)

    @pl.loop(0, CPW)
    def _(c):
        base = (wid * CPW + c) * C
        pltpu.sync_copy(src_hbm.at[pl.ds(base, C)], src_v.at[0])
        pltpu.sync_copy(dst_hbm.at[pl.ds(base, C)], dst_v.at[0])
        pltpu.sync_copy(w_hbm.at[pl.ds(base, C)], w_v)
        for g in range(C // 16):
            s16 = src_v[0, pl.ds(g * 16, 16)]
            d16 = dst_v[0, pl.ds(g * 16, 16)]
            w16 = w_v[pl.ds(g * 16, 16)]
            nv = plsc.load_gather(dinv_v, [s16]) * w16 * plsc.load_gather(dinv_v, [d16])
            nb_v[pl.ds(g * 16, 16)] = nv
        pltpu.sync_copy(nb_v, out_hbm.at[pl.ds(base, C)])


@functools.partial(
    pl.kernel,
    out_type=jax.ShapeDtypeStruct((NC, N, D), jnp.float32),
    mesh=_mesh,
    scratch_types=[
        pltpu.VMEM((1, C), jnp.int32),
        pltpu.VMEM((1, C), jnp.int32),
        pltpu.VMEM((C,), jnp.float32),
        pltpu.VMEM((C, D), jnp.float32),
        pltpu.VMEM((RPW // 5, D), jnp.float32),
        pltpu.VMEM_SHARED((N, D), jnp.float32),
        pltpu.SemaphoreType.DMA,
    ],
)
def _scatter_kernel(h_hbm, src_hbm, dst_hbm, norm_hbm, out_hbm,
                    src_v, dst_v, nb_v, rows_v, zb_v, acc_sh, sem):
    cid = lax.axis_index("c")
    sid = lax.axis_index("s")
    wid = cid * NS + sid
    zeros16 = jnp.zeros((16,), jnp.float32)
    ZR = RPW // 5

    @pl.loop(0, ZR)
    def _(r):
        for k in range(D // 16):
            zb_v[r, pl.ds(k * 16, 16)] = zeros16

    @pl.loop(0, 5)
    def _(k):
        pltpu.sync_copy(zb_v, acc_sh.at[pl.ds(sid * RPW + k * ZR, ZR)])

    plsc.subcore_barrier()

    @pl.loop(0, CPW)
    def _(c):
        base = (wid * CPW + c) * C
        pltpu.sync_copy(src_hbm.at[pl.ds(base, C)], src_v.at[0])
        pltpu.sync_copy(dst_hbm.at[pl.ds(base, C)], dst_v.at[0])
        pltpu.sync_copy(norm_hbm.at[pl.ds(base, C)], nb_v)
        pltpu.async_copy(h_hbm.at[src_v.at[0]], rows_v, sem).wait()

        @pl.loop(0, C)
        def _(e):
            nv = plsc.load_gather(nb_v, [jnp.broadcast_to(e, (16,))])
            for k in range(D // 16):
                rows_v[e, pl.ds(k * 16, 16)] = rows_v[e, pl.ds(k * 16, 16)] * nv

        pltpu.sync_copy(rows_v, acc_sh.at[dst_v.at[0]], add=True)

    plsc.subcore_barrier()

    @pl.loop(0, 5)
    def _(k):
        off = sid * RPW + k * ZR
        pltpu.sync_copy(acc_sh.at[pl.ds(off, ZR)],
                        out_hbm.at[cid].at[pl.ds(off, ZR)])


def _tc_prep_body(parts_ref, x_ref, w_ref, dinv_ref, t_ref):
    deg = jnp.sum(parts_ref[...], axis=(0, 2)) + 1.0
    dinv = jnp.where(deg > 0.0, lax.rsqrt(jnp.abs(deg) + 1e-30), 0.0)
    dinv_ref[...] = dinv[:, None]
    t_ref[...] = jnp.dot(x_ref[...], w_ref[...],
                         preferred_element_type=jnp.float32)


def _tc_prep(parts, x, w):
    return pl.pallas_call(
        _tc_prep_body,
        out_shape=[
            jax.ShapeDtypeStruct((N, 1), jnp.float32),
            jax.ShapeDtypeStruct((N, D), jnp.float32),
        ],
    )(parts, x, w)


def _tc_fuse_body(p_ref, t_ref, dinv_ref, b_ref, w_ref, out_ref):
    d2 = dinv_ref[...] * dinv_ref[...]
    agg = p_ref[0] + p_ref[1] + d2 * t_ref[...] + b_ref[...]
    h = jnp.maximum(agg, 0.0)
    out_ref[...] = jnp.dot(h, w_ref[...], preferred_element_type=jnp.float32)


def _tc_fuse(p, t, dinv, b, w):
    return pl.pallas_call(
        _tc_fuse_body,
        out_shape=jax.ShapeDtypeStruct((N, D), jnp.float32),
    )(p, t, dinv, b, w)


def _tc_final_body(p_ref, t_ref, dinv_ref, b_ref, batch_ref, wl_ref, bl_ref,
                   out_ref):
    d2 = dinv_ref[...] * dinv_ref[...]
    h3 = p_ref[0] + p_ref[1] + d2 * t_ref[...] + b_ref[...]
    gid = lax.broadcasted_iota(jnp.int32, (G, N), 0)
    onehot = (batch_ref[...] == gid).astype(jnp.float32)
    sums = jnp.dot(onehot, h3, preferred_element_type=jnp.float32)
    counts = jnp.sum(onehot, axis=1, keepdims=True)
    hg = sums / jnp.maximum(counts, 1.0)
    out_ref[...] = jnp.dot(hg, wl_ref[...],
                           preferred_element_type=jnp.float32) + bl_ref[...]


def _tc_final(p, t, dinv, b, batch2d, wl, bl):
    return pl.pallas_call(
        _tc_final_body,
        out_shape=jax.ShapeDtypeStruct((G, DO), jnp.float32),
    )(p, t, dinv, b, batch2d, wl, bl)


def kernel(x, edge_index, edge_weight, batch, W1, b1, W2, b2, W3, b3, Wl, bl):
    src = edge_index[0]
    dst = edge_index[1]
    npad = E_PAD - E
    pad_idx = (jnp.arange(npad, dtype=jnp.int32) % N)
    src_p = jnp.concatenate([src.astype(jnp.int32), pad_idx])
    dst_p = jnp.concatenate([dst.astype(jnp.int32), pad_idx])
    w_p = jnp.concatenate([edge_weight, jnp.zeros((npad,), jnp.float32)])

    parts = _deg_kernel(dst_p, w_p)
    dinv2d, t1 = _tc_prep(parts, x, W1)
    norm = _norm_kernel(src_p, dst_p, w_p, dinv2d.reshape(N))

    p1 = _scatter_kernel(t1, src_p, dst_p, norm)
    t2 = _tc_fuse(p1, t1, dinv2d, b1.reshape(1, D), W2)
    p2 = _scatter_kernel(t2, src_p, dst_p, norm)
    t3 = _tc_fuse(p2, t2, dinv2d, b2.reshape(1, D), W3)
    p3 = _scatter_kernel(t3, src_p, dst_p, norm)

    return _tc_final(p3, t3, dinv2d, b3.reshape(1, D),
                     batch.reshape(1, N).astype(jnp.int32),
                     Wl, bl.reshape(1, DO))


# speedup vs baseline: 8.1052x; 8.1052x over previous
import dataclasses
import functools

import jax
import jax.numpy as jnp
from jax import lax
from jax.experimental import pallas as pl
from jax.experimental.pallas import tpu as pltpu
from jax.experimental.pallas import tpu_sc as plsc

N = 10000
E = 320000
D = 128
G = 64
DO = 10

NC = 2
NS = 16
NW = NC * NS
C = 128
CPW = 79
E_PAD = NW * CPW * C
N_PAD = 10240
RPW = N_PAD // NS

_mesh = plsc.VectorSubcoreMesh(core_axis_name="c", subcore_axis_name="s")

_sc_params = pltpu.CompilerParams()
if "needs_layout_passes" in pltpu.CompilerParams.__dataclass_fields__:
    _sc_params = dataclasses.replace(_sc_params, needs_layout_passes=False)


@functools.partial(
    pl.kernel,
    out_type=jax.ShapeDtypeStruct((NC, N_PAD, D), jnp.float32),
    mesh=_mesh,
    compiler_params=_sc_params,
    scratch_types=[
        pltpu.VMEM((C,), jnp.int32),
        pltpu.VMEM((C,), jnp.float32),
        pltpu.VMEM((C, D), jnp.float32),
        pltpu.VMEM_SHARED((N_PAD, D), jnp.float32),
        pltpu.SemaphoreType.DMA,
    ],
)
def _deg_kernel(dst_hbm, w_hbm, zeros_hbm, out_hbm, dst_v, w_v, val_v, acc_sh,
                sem):
    del sem
    cid = lax.axis_index("c")
    sid = lax.axis_index("s")
    wid = cid * NS + sid
    zeros16 = jnp.zeros((16,), jnp.float32)

    @pl.loop(0, C)
    def _(r):
        for k in range(D // 16):
            val_v[r, pl.ds(k * 16, 16)] = zeros16

    @pl.when(sid == 0)
    def _():
        pltpu.sync_copy(zeros_hbm, acc_sh)

    plsc.subcore_barrier()
    onehot0 = (lax.iota(jnp.int32, 16) == 0).astype(jnp.float32)

    @pl.loop(0, CPW)
    def _(c):
        base = (wid * CPW + c) * C
        pltpu.sync_copy(dst_hbm.at[pl.ds(base, C)], dst_v)
        pltpu.sync_copy(w_hbm.at[pl.ds(base, C)], w_v)

        @pl.loop(0, C)
        def _(e):
            wv = plsc.load_gather(w_v, [jnp.broadcast_to(e, (16,))])
            val_v[e, pl.ds(0, 16)] = wv * onehot0

        pltpu.sync_copy(val_v, acc_sh.at[dst_v], add=True)

    plsc.subcore_barrier()

    @pl.when(sid == 0)
    def _():
        pltpu.sync_copy(acc_sh, out_hbm.at[cid])


@functools.partial(
    pl.kernel,
    out_type=jax.ShapeDtypeStruct((E_PAD,), jnp.float32),
    mesh=_mesh,
    compiler_params=_sc_params,
    scratch_types=[
        pltpu.VMEM((N,), jnp.float32),
        pltpu.VMEM((C,), jnp.int32),
        pltpu.VMEM((C,), jnp.int32),
        pltpu.VMEM((C,), jnp.float32),
        pltpu.VMEM((C,), jnp.float32),
    ],
)
def _norm_kernel(src_hbm, dst_hbm, w_hbm, dinv_hbm, out_hbm,
                 dinv_v, src_v, dst_v, w_v, nb_v):
    cid = lax.axis_index("c")
    sid = lax.axis_index("s")
    wid = cid * NS + sid
    pltpu.sync_copy(dinv_hbm, dinv_v)

    @pl.loop(0, CPW)
    def _(c):
        base = (wid * CPW + c) * C
        pltpu.sync_copy(src_hbm.at[pl.ds(base, C)], src_v)
        pltpu.sync_copy(dst_hbm.at[pl.ds(base, C)], dst_v)
        pltpu.sync_copy(w_hbm.at[pl.ds(base, C)], w_v)
        for g in range(C // 16):
            s16 = src_v[pl.ds(g * 16, 16)]
            d16 = dst_v[pl.ds(g * 16, 16)]
            w16 = w_v[pl.ds(g * 16, 16)]
            nv = plsc.load_gather(dinv_v, [s16]) * w16 * plsc.load_gather(dinv_v, [d16])
            nb_v[pl.ds(g * 16, 16)] = nv
        pltpu.sync_copy(nb_v, out_hbm.at[pl.ds(base, C)])


@functools.partial(
    pl.kernel,
    out_type=jax.ShapeDtypeStruct((NC, N_PAD, D), jnp.float32),
    mesh=_mesh,
    compiler_params=_sc_params,
    scratch_types=[
        pltpu.VMEM((C,), jnp.int32),
        pltpu.VMEM((C,), jnp.int32),
        pltpu.VMEM((C,), jnp.float32),
        pltpu.VMEM((C, D), jnp.float32),
        pltpu.VMEM_SHARED((N_PAD, D), jnp.float32),
        pltpu.SemaphoreType.DMA,
    ],
)
def _scatter_kernel(h_hbm, src_hbm, dst_hbm, norm_hbm, zeros_hbm, out_hbm,
                    src_v, dst_v, nb_v, rows_v, acc_sh, sem):
    cid = lax.axis_index("c")
    sid = lax.axis_index("s")
    wid = cid * NS + sid

    @pl.when(sid == 0)
    def _():
        pltpu.sync_copy(zeros_hbm, acc_sh)

    plsc.subcore_barrier()

    @pl.loop(0, CPW)
    def _(c):
        base = (wid * CPW + c) * C
        pltpu.sync_copy(src_hbm.at[pl.ds(base, C)], src_v)
        pltpu.sync_copy(dst_hbm.at[pl.ds(base, C)], dst_v)
        pltpu.sync_copy(norm_hbm.at[pl.ds(base, C)], nb_v)
        pltpu.async_copy(h_hbm.at[src_v], rows_v, sem).wait()

        @pl.loop(0, C)
        def _(e):
            nv = plsc.load_gather(nb_v, [jnp.broadcast_to(e, (16,))])
            for k in range(D // 16):
                rows_v[e, pl.ds(k * 16, 16)] = rows_v[e, pl.ds(k * 16, 16)] * nv

        pltpu.sync_copy(rows_v, acc_sh.at[dst_v], add=True)

    plsc.subcore_barrier()

    @pl.when(sid == 0)
    def _():
        pltpu.sync_copy(acc_sh, out_hbm.at[cid])


def _tc_prep_body(parts_ref, x_ref, w_ref, dinv_ref, t_ref):
    deg = parts_ref[0, :N, 0:1] + parts_ref[1, :N, 0:1] + 1.0
    dinv = jnp.where(deg > 0.0, lax.rsqrt(jnp.abs(deg) + 1e-30), 0.0)
    dinv_ref[...] = dinv
    t_ref[...] = jnp.dot(x_ref[...], w_ref[...],
                         preferred_element_type=jnp.float32)


def _tc_prep(parts, x, w):
    return pl.pallas_call(
        _tc_prep_body,
        out_shape=[
            jax.ShapeDtypeStruct((N, 1), jnp.float32),
            jax.ShapeDtypeStruct((N, D), jnp.float32),
        ],
    )(parts, x, w)


def _tc_fuse_body(p_ref, t_ref, dinv_ref, b_ref, w_ref, out_ref):
    d2 = dinv_ref[...] * dinv_ref[...]
    agg = p_ref[0, :N] + p_ref[1, :N] + d2 * t_ref[...] + b_ref[...]
    h = jnp.maximum(agg, 0.0)
    out_ref[...] = jnp.dot(h, w_ref[...], preferred_element_type=jnp.float32)


def _tc_fuse(p, t, dinv, b, w):
    return pl.pallas_call(
        _tc_fuse_body,
        out_shape=jax.ShapeDtypeStruct((N, D), jnp.float32),
    )(p, t, dinv, b, w)


def _tc_final_body(p_ref, t_ref, dinv_ref, b_ref, batch_ref, wl_ref, bl_ref,
                   out_ref):
    d2 = dinv_ref[...] * dinv_ref[...]
    h3 = p_ref[0, :N] + p_ref[1, :N] + d2 * t_ref[...] + b_ref[...]
    gid = lax.broadcasted_iota(jnp.int32, (G, N), 0)
    onehot = (batch_ref[...] == gid).astype(jnp.float32)
    sums = jnp.dot(onehot, h3, preferred_element_type=jnp.float32)
    counts = jnp.sum(onehot, axis=1, keepdims=True)
    hg = sums / jnp.maximum(counts, 1.0)
    out_ref[...] = jnp.dot(hg, wl_ref[...],
                           preferred_element_type=jnp.float32) + bl_ref[...]


def _tc_final(p, t, dinv, b, batch2d, wl, bl):
    return pl.pallas_call(
        _tc_final_body,
        out_shape=jax.ShapeDtypeStruct((G, DO), jnp.float32),
    )(p, t, dinv, b, batch2d, wl, bl)


def kernel(x, edge_index, edge_weight, batch, W1, b1, W2, b2, W3, b3, Wl, bl):
    src = edge_index[0]
    dst = edge_index[1]
    npad = E_PAD - E
    pad_idx = (jnp.arange(npad, dtype=jnp.int32) % N)
    src_p = jnp.concatenate([src.astype(jnp.int32), pad_idx])
    dst_p = jnp.concatenate([dst.astype(jnp.int32), pad_idx])
    w_p = jnp.concatenate([edge_weight, jnp.zeros((npad,), jnp.float32)])

    zeros_nd = jnp.zeros((N_PAD, D), jnp.float32)
    parts = _deg_kernel(dst_p, w_p, zeros_nd)
    dinv2d, t1 = _tc_prep(parts, x, W1)
    norm = _norm_kernel(src_p, dst_p, w_p, dinv2d.reshape(N))

    p1 = _scatter_kernel(t1, src_p, dst_p, norm, zeros_nd)
    t2 = _tc_fuse(p1, t1, dinv2d, b1.reshape(1, D), W2)
    p2 = _scatter_kernel(t2, src_p, dst_p, norm, zeros_nd)
    t3 = _tc_fuse(p2, t2, dinv2d, b2.reshape(1, D), W3)
    p3 = _scatter_kernel(t3, src_p, dst_p, norm, zeros_nd)

    return _tc_final(p3, t3, dinv2d, b3.reshape(1, D),
                     batch.reshape(1, N).astype(jnp.int32),
                     Wl, bl.reshape(1, DO))


# speedup vs baseline: 20.5288x; 2.5328x over previous
import dataclasses
import functools

import jax
import jax.numpy as jnp
from jax import lax
from jax.experimental import pallas as pl
from jax.experimental.pallas import tpu as pltpu
from jax.experimental.pallas import tpu_sc as plsc

N = 10000
E = 320000
D = 128
G = 64
DO = 10

NC = 2
NS = 16
NW = NC * NS
C = 128
CPW = 79
E_PAD = NW * CPW * C
N_PAD = N

_mesh = plsc.VectorSubcoreMesh(core_axis_name="c", subcore_axis_name="s")

_sc_params = pltpu.CompilerParams()
if "needs_layout_passes" in pltpu.CompilerParams.__dataclass_fields__:
    _sc_params = dataclasses.replace(_sc_params, needs_layout_passes=False)


@functools.partial(
    pl.kernel,
    out_type=jax.ShapeDtypeStruct((NC, N_PAD, D), jnp.float32),
    mesh=_mesh,
    compiler_params=_sc_params,
    scratch_types=[
        pltpu.VMEM((C, D), jnp.float32),
        pltpu.VMEM((C, D), jnp.float32),
        pltpu.VMEM((C,), jnp.int32),
        pltpu.VMEM((C,), jnp.int32),
        pltpu.VMEM((C,), jnp.float32),
        pltpu.VMEM((C,), jnp.float32),
        pltpu.VMEM_SHARED((N_PAD, D), jnp.float32),
        pltpu.SemaphoreType.DMA,
        pltpu.SemaphoreType.DMA,
        pltpu.SemaphoreType.DMA,
        pltpu.SemaphoreType.DMA,
    ],
)
def _deg_kernel(dst_hbm, w_hbm, zeros_hbm, out_hbm,
                v0, v1, d0, d1, w0, w1, acc_sh, s0, s1, iw0, iw1):
    cid = lax.axis_index("c")
    sid = lax.axis_index("s")
    wid = cid * NS + sid
    val = (v0, v1)
    dstv = (d0, d1)
    wv = (w0, w1)
    ssem = (s0, s1)
    iwsem = (iw0, iw1)
    zeros16 = jnp.zeros((16,), jnp.float32)
    onehot0 = (lax.iota(jnp.int32, 16) == 0).astype(jnp.float32)

    for p in range(2):
        @pl.loop(0, C)
        def _(r, p=p):
            for k in range(D // 16):
                val[p][r, pl.ds(k * 16, 16)] = zeros16

    @pl.when(sid == 0)
    def _():
        pltpu.sync_copy(zeros_hbm, acc_sh)

    plsc.subcore_barrier()

    def istart_w(i, p):
        pltpu.async_copy(w_hbm.at[pl.ds((wid * CPW + i) * C, C)], wv[p],
                         iwsem[p])

    def iwait_w(p):
        pltpu.make_async_copy(w_hbm.at[pl.ds(0, C)], wv[p], iwsem[p]).wait()

    def swait(p):
        pltpu.make_async_copy(val[p], acc_sh.at[dstv[p]], ssem[p]).wait()

    def body(i, p, do_swait, prefetch):
        if do_swait:
            swait(p)
        pltpu.sync_copy(dst_hbm.at[pl.ds((wid * CPW + i) * C, C)], dstv[p])
        iwait_w(p)

        @pl.loop(0, C)
        def _(e):
            w16 = plsc.load_gather(wv[p], [jnp.broadcast_to(e, (16,))])
            val[p][e, pl.ds(0, 16)] = w16 * onehot0

        pltpu.async_copy(val[p], acc_sh.at[dstv[p]], ssem[p], add=True)
        if prefetch:
            istart_w(i + 2, p)

    istart_w(0, 0)
    istart_w(1, 1)
    body(0, 0, False, True)
    body(1, 1, False, True)

    @pl.loop(0, (CPW - 5) // 2)
    def _(k):
        body(2 * k + 2, 0, True, True)
        body(2 * k + 3, 1, True, True)

    body(CPW - 3, 0, True, True)
    body(CPW - 2, 1, True, False)
    body(CPW - 1, 0, True, False)
    swait(1)
    swait(0)

    plsc.subcore_barrier()

    @pl.when(sid == 0)
    def _():
        pltpu.sync_copy(acc_sh, out_hbm.at[cid])


@functools.partial(
    pl.kernel,
    out_type=jax.ShapeDtypeStruct((E_PAD,), jnp.float32),
    mesh=_mesh,
    compiler_params=_sc_params,
    scratch_types=[
        pltpu.VMEM((N,), jnp.float32),
        pltpu.VMEM((CPW * C,), jnp.int32),
        pltpu.VMEM((CPW * C,), jnp.int32),
        pltpu.VMEM((CPW * C,), jnp.float32),
        pltpu.VMEM((CPW * C,), jnp.float32),
    ],
)
def _norm_kernel(src_hbm, dst_hbm, w_hbm, dinv_hbm, out_hbm,
                 dinv_v, src_all, dst_all, w_all, nb_all):
    cid = lax.axis_index("c")
    sid = lax.axis_index("s")
    wid = cid * NS + sid
    ebase = wid * (CPW * C)
    pltpu.sync_copy(dinv_hbm, dinv_v)
    pltpu.sync_copy(src_hbm.at[pl.ds(ebase, CPW * C)], src_all)
    pltpu.sync_copy(dst_hbm.at[pl.ds(ebase, CPW * C)], dst_all)
    pltpu.sync_copy(w_hbm.at[pl.ds(ebase, CPW * C)], w_all)

    @pl.loop(0, CPW * C // 16)
    def _(g):
        s16 = src_all[pl.ds(g * 16, 16)]
        d16 = dst_all[pl.ds(g * 16, 16)]
        w16 = w_all[pl.ds(g * 16, 16)]
        nb_all[pl.ds(g * 16, 16)] = (plsc.load_gather(dinv_v, [s16]) * w16
                                     * plsc.load_gather(dinv_v, [d16]))

    pltpu.sync_copy(nb_all, out_hbm.at[pl.ds(ebase, CPW * C)])


@functools.partial(
    pl.kernel,
    out_type=jax.ShapeDtypeStruct((NC, N_PAD, D), jnp.float32),
    mesh=_mesh,
    compiler_params=_sc_params,
    scratch_types=[
        pltpu.VMEM((C, D), jnp.float32),
        pltpu.VMEM((C, D), jnp.float32),
        pltpu.VMEM((C, D), jnp.float32),
        pltpu.VMEM((C,), jnp.int32),
        pltpu.VMEM((C,), jnp.int32),
        pltpu.VMEM((C,), jnp.int32),
        pltpu.VMEM((C,), jnp.int32),
        pltpu.VMEM((C,), jnp.int32),
        pltpu.VMEM((C,), jnp.int32),
        pltpu.VMEM((C,), jnp.float32),
        pltpu.VMEM((C,), jnp.float32),
        pltpu.VMEM((C,), jnp.float32),
        pltpu.VMEM_SHARED((N_PAD, D), jnp.float32),
        pltpu.SemaphoreType.DMA,
        pltpu.SemaphoreType.DMA,
        pltpu.SemaphoreType.DMA,
        pltpu.SemaphoreType.DMA,
        pltpu.SemaphoreType.DMA,
        pltpu.SemaphoreType.DMA,
        pltpu.SemaphoreType.DMA,
        pltpu.SemaphoreType.DMA,
        pltpu.SemaphoreType.DMA,
        pltpu.SemaphoreType.DMA,
        pltpu.SemaphoreType.DMA,
        pltpu.SemaphoreType.DMA,
        pltpu.SemaphoreType.DMA,
        pltpu.SemaphoreType.DMA,
        pltpu.SemaphoreType.DMA,
    ],
)
def _scatter_kernel(h_hbm, src_hbm, dst_hbm, norm_hbm, zeros_hbm, out_hbm,
                    r0, r1, r2, sv0, sv1, sv2, dv0, dv1, dv2, nv0, nv1, nv2,
                    acc_sh,
                    g0, g1, g2, s0, s1, s2,
                    is0, is1, is2, id0, id1, id2, in0, in1, in2):
    cid = lax.axis_index("c")
    sid = lax.axis_index("s")
    wid = cid * NS + sid
    rows = (r0, r1, r2)
    srcv = (sv0, sv1, sv2)
    dstv = (dv0, dv1, dv2)
    normv = (nv0, nv1, nv2)
    gsem = (g0, g1, g2)
    ssem = (s0, s1, s2)
    isem = (is0, is1, is2)
    idsem = (id0, id1, id2)
    insem = (in0, in1, in2)

    @pl.when(sid == 0)
    def _():
        pltpu.sync_copy(zeros_hbm, acc_sh)

    plsc.subcore_barrier()

    def chunk(i):
        return pl.ds((wid * CPW + i) * C, C)

    def istart_src(i, p):
        pltpu.async_copy(src_hbm.at[chunk(i)], srcv[p], isem[p])

    def iwait_src(p):
        pltpu.make_async_copy(src_hbm.at[chunk(0)], srcv[p], isem[p]).wait()

    def istart_dst(i, p):
        pltpu.async_copy(dst_hbm.at[chunk(i)], dstv[p], idsem[p])

    def iwait_dst(p):
        pltpu.make_async_copy(dst_hbm.at[chunk(0)], dstv[p], idsem[p]).wait()

    def istart_norm(i, p):
        pltpu.async_copy(norm_hbm.at[chunk(i)], normv[p], insem[p])

    def iwait_norm(p):
        pltpu.make_async_copy(norm_hbm.at[chunk(0)], normv[p], insem[p]).wait()

    def gstart(p):
        pltpu.async_copy(h_hbm.at[srcv[p]], rows[p], gsem[p])

    def gwait(p):
        pltpu.make_async_copy(h_hbm.at[srcv[p]], rows[p], gsem[p]).wait()

    def swait(p):
        pltpu.make_async_copy(rows[p], acc_sh.at[dstv[p]], ssem[p]).wait()

    def body(i, p, first, last):
        p2 = (p + 2) % 3
        if not last:
            istart_src(i + 2, p2)
            istart_norm(i + 2, p2)
        gwait(p)
        iwait_norm(p)

        @pl.loop(0, C)
        def _(e):
            nv = plsc.load_gather(normv[p], [jnp.broadcast_to(e, (16,))])
            for k in range(D // 16):
                rows[p][e, pl.ds(k * 16, 16)] = rows[p][e, pl.ds(k * 16, 16)] * nv

        iwait_dst(p)
        pltpu.async_copy(rows[p], acc_sh.at[dstv[p]], ssem[p], add=True)
        if not first:
            swait(p2)
        if not last:
            istart_dst(i + 2, p2)
            iwait_src(p2)
            gstart(p2)

    for p in range(2):
        istart_src(p, p)
        istart_norm(p, p)
        istart_dst(p, p)
    iwait_src(0)
    gstart(0)
    iwait_src(1)
    gstart(1)
    body(0, 0, True, False)
    body(1, 1, False, False)

    @pl.loop(0, (CPW - 4) // 3)
    def _(k):
        i = 3 * k + 2
        body(i, 2, False, False)
        body(i + 1, 0, False, False)
        body(i + 2, 1, False, False)

    body(CPW - 2, 2, False, True)
    body(CPW - 1, 0, False, True)
    swait(0)

    plsc.subcore_barrier()

    @pl.when(sid == 0)
    def _():
        pltpu.sync_copy(acc_sh, out_hbm.at[cid])


def _tc_prep_body(parts_ref, x_ref, w_ref, dinv_ref, t_ref):
    deg = parts_ref[0, :N, 0:1] + parts_ref[1, :N, 0:1] + 1.0
    dinv = jnp.where(deg > 0.0, lax.rsqrt(jnp.abs(deg) + 1e-30), 0.0)
    dinv_ref[...] = dinv
    t_ref[...] = jnp.dot(x_ref[...], w_ref[...],
                         preferred_element_type=jnp.float32)


def _tc_prep(parts, x, w):
    return pl.pallas_call(
        _tc_prep_body,
        out_shape=[
            jax.ShapeDtypeStruct((N, 1), jnp.float32),
            jax.ShapeDtypeStruct((N, D), jnp.float32),
        ],
    )(parts, x, w)


def _tc_fuse_body(p_ref, t_ref, dinv_ref, b_ref, w_ref, out_ref):
    d2 = dinv_ref[...] * dinv_ref[...]
    agg = p_ref[0, :N] + p_ref[1, :N] + d2 * t_ref[...] + b_ref[...]
    h = jnp.maximum(agg, 0.0)
    out_ref[...] = jnp.dot(h, w_ref[...], preferred_element_type=jnp.float32)


def _tc_fuse(p, t, dinv, b, w):
    return pl.pallas_call(
        _tc_fuse_body,
        out_shape=jax.ShapeDtypeStruct((N, D), jnp.float32),
    )(p, t, dinv, b, w)


def _tc_final_body(p_ref, t_ref, dinv_ref, b_ref, batch_ref, wl_ref, bl_ref,
                   out_ref):
    d2 = dinv_ref[...] * dinv_ref[...]
    h3 = p_ref[0, :N] + p_ref[1, :N] + d2 * t_ref[...] + b_ref[...]
    gid = lax.broadcasted_iota(jnp.int32, (G, N), 0)
    onehot = (batch_ref[...] == gid).astype(jnp.float32)
    sums = jnp.dot(onehot, h3, preferred_element_type=jnp.float32)
    counts = jnp.sum(onehot, axis=1, keepdims=True)
    hg = sums / jnp.maximum(counts, 1.0)
    out_ref[...] = jnp.dot(hg, wl_ref[...],
                           preferred_element_type=jnp.float32) + bl_ref[...]


def _tc_final(p, t, dinv, b, batch2d, wl, bl):
    return pl.pallas_call(
        _tc_final_body,
        out_shape=jax.ShapeDtypeStruct((G, DO), jnp.float32),
    )(p, t, dinv, b, batch2d, wl, bl)


def kernel(x, edge_index, edge_weight, batch, W1, b1, W2, b2, W3, b3, Wl, bl):
    src = edge_index[0]
    dst = edge_index[1]
    npad = E_PAD - E
    pad_idx = (jnp.arange(npad, dtype=jnp.int32) % N)
    src_p = jnp.concatenate([src.astype(jnp.int32), pad_idx])
    dst_p = jnp.concatenate([dst.astype(jnp.int32), pad_idx])
    w_p = jnp.concatenate([edge_weight, jnp.zeros((npad,), jnp.float32)])

    zeros_nd = jnp.zeros((N_PAD, D), jnp.float32)
    parts = _deg_kernel(dst_p, w_p, zeros_nd)
    dinv2d, t1 = _tc_prep(parts, x, W1)
    norm = _norm_kernel(src_p, dst_p, w_p, dinv2d.reshape(N))

    p1 = _scatter_kernel(t1, src_p, dst_p, norm, zeros_nd)
    t2 = _tc_fuse(p1, t1, dinv2d, b1.reshape(1, D), W2)
    p2 = _scatter_kernel(t2, src_p, dst_p, norm, zeros_nd)
    t3 = _tc_fuse(p2, t2, dinv2d, b2.reshape(1, D), W3)
    p3 = _scatter_kernel(t3, src_p, dst_p, norm, zeros_nd)

    return _tc_final(p3, t3, dinv2d, b3.reshape(1, D),
                     batch.reshape(1, N).astype(jnp.int32),
                     Wl, bl.reshape(1, DO))


# speedup vs baseline: 23.7188x; 1.1554x over previous
import dataclasses
import functools

import jax
import jax.numpy as jnp
from jax import lax
from jax.experimental import pallas as pl
from jax.experimental.pallas import tpu as pltpu
from jax.experimental.pallas import tpu_sc as plsc

N = 10000
E = 320000
D = 128
G = 64
DO = 10

NC = 2
NS = 16
NW = NC * NS
C = 128
CPW = 79
E_PAD = NW * CPW * C
N_PAD = N

_mesh = plsc.VectorSubcoreMesh(core_axis_name="c", subcore_axis_name="s")

_sc_params = pltpu.CompilerParams()
if "needs_layout_passes" in pltpu.CompilerParams.__dataclass_fields__:
    _sc_params = dataclasses.replace(_sc_params, needs_layout_passes=False)


@functools.partial(
    pl.kernel,
    out_type=jax.ShapeDtypeStruct((NC, N_PAD, D), jnp.float32),
    mesh=_mesh,
    compiler_params=_sc_params,
    scratch_types=[
        pltpu.VMEM((C, D), jnp.float32),
        pltpu.VMEM((C, D), jnp.float32),
        pltpu.VMEM((C,), jnp.int32),
        pltpu.VMEM((C,), jnp.int32),
        pltpu.VMEM((C,), jnp.float32),
        pltpu.VMEM((C,), jnp.float32),
        pltpu.VMEM_SHARED((N_PAD, D), jnp.float32),
        pltpu.SemaphoreType.DMA,
        pltpu.SemaphoreType.DMA,
        pltpu.SemaphoreType.DMA,
        pltpu.SemaphoreType.DMA,
    ],
)
def _deg_kernel(dst_hbm, w_hbm, zeros_hbm, out_hbm,
                v0, v1, d0, d1, w0, w1, acc_sh, s0, s1, iw0, iw1):
    cid = lax.axis_index("c")
    sid = lax.axis_index("s")
    wid = cid * NS + sid
    val = (v0, v1)
    dstv = (d0, d1)
    wv = (w0, w1)
    ssem = (s0, s1)
    iwsem = (iw0, iw1)
    zeros16 = jnp.zeros((16,), jnp.float32)
    onehot0 = (lax.iota(jnp.int32, 16) == 0).astype(jnp.float32)

    for p in range(2):
        @pl.loop(0, C)
        def _(r, p=p):
            for k in range(D // 16):
                val[p][r, pl.ds(k * 16, 16)] = zeros16

    @pl.when(sid == 0)
    def _():
        pltpu.sync_copy(zeros_hbm, acc_sh)

    plsc.subcore_barrier()

    def istart_w(i, p):
        pltpu.async_copy(w_hbm.at[pl.ds((wid * CPW + i) * C, C)], wv[p],
                         iwsem[p])

    def iwait_w(p):
        pltpu.make_async_copy(w_hbm.at[pl.ds(0, C)], wv[p], iwsem[p]).wait()

    def swait(p):
        pltpu.make_async_copy(val[p], acc_sh.at[dstv[p]], ssem[p]).wait()

    def body(i, p, do_swait, prefetch):
        if do_swait:
            swait(p)
        pltpu.sync_copy(dst_hbm.at[pl.ds((wid * CPW + i) * C, C)], dstv[p])
        iwait_w(p)

        @plsc.parallel_loop(0, C, unroll=4)
        def _(e):
            w16 = plsc.load_gather(wv[p], [jnp.broadcast_to(e, (16,))])
            val[p][e, pl.ds(0, 16)] = w16 * onehot0

        pltpu.async_copy(val[p], acc_sh.at[dstv[p]], ssem[p], add=True)
        if prefetch:
            istart_w(i + 2, p)

    istart_w(0, 0)
    istart_w(1, 1)
    body(0, 0, False, True)
    body(1, 1, False, True)

    @pl.loop(0, (CPW - 5) // 2)
    def _(k):
        body(2 * k + 2, 0, True, True)
        body(2 * k + 3, 1, True, True)

    body(CPW - 3, 0, True, True)
    body(CPW - 2, 1, True, False)
    body(CPW - 1, 0, True, False)
    swait(1)
    swait(0)

    plsc.subcore_barrier()

    @pl.when(sid == 0)
    def _():
        pltpu.sync_copy(acc_sh, out_hbm.at[cid])


@functools.partial(
    pl.kernel,
    out_type=jax.ShapeDtypeStruct((E_PAD,), jnp.float32),
    mesh=_mesh,
    compiler_params=_sc_params,
    scratch_types=[
        pltpu.VMEM((N,), jnp.float32),
        pltpu.VMEM((CPW * C,), jnp.int32),
        pltpu.VMEM((CPW * C,), jnp.int32),
        pltpu.VMEM((CPW * C,), jnp.float32),
        pltpu.VMEM((CPW * C,), jnp.float32),
    ],
)
def _norm_kernel(src_hbm, dst_hbm, w_hbm, dinv_hbm, out_hbm,
                 dinv_v, src_all, dst_all, w_all, nb_all):
    cid = lax.axis_index("c")
    sid = lax.axis_index("s")
    wid = cid * NS + sid
    ebase = wid * (CPW * C)
    pltpu.sync_copy(dinv_hbm, dinv_v)
    pltpu.sync_copy(src_hbm.at[pl.ds(ebase, CPW * C)], src_all)
    pltpu.sync_copy(dst_hbm.at[pl.ds(ebase, CPW * C)], dst_all)
    pltpu.sync_copy(w_hbm.at[pl.ds(ebase, CPW * C)], w_all)

    @plsc.parallel_loop(0, CPW * C // 16, unroll=4)
    def _(g):
        s16 = src_all[pl.ds(g * 16, 16)]
        d16 = dst_all[pl.ds(g * 16, 16)]
        w16 = w_all[pl.ds(g * 16, 16)]
        nb_all[pl.ds(g * 16, 16)] = (plsc.load_gather(dinv_v, [s16]) * w16
                                     * plsc.load_gather(dinv_v, [d16]))

    pltpu.sync_copy(nb_all, out_hbm.at[pl.ds(ebase, CPW * C)])


@functools.partial(
    pl.kernel,
    out_type=jax.ShapeDtypeStruct((NC, N_PAD, D), jnp.float32),
    mesh=_mesh,
    compiler_params=_sc_params,
    scratch_types=[
        pltpu.VMEM((C, D), jnp.float32),
        pltpu.VMEM((C, D), jnp.float32),
        pltpu.VMEM((C, D), jnp.float32),
        pltpu.VMEM((C,), jnp.int32),
        pltpu.VMEM((C,), jnp.int32),
        pltpu.VMEM((C,), jnp.int32),
        pltpu.VMEM((C,), jnp.int32),
        pltpu.VMEM((C,), jnp.int32),
        pltpu.VMEM((C,), jnp.int32),
        pltpu.VMEM((C,), jnp.float32),
        pltpu.VMEM((C,), jnp.float32),
        pltpu.VMEM((C,), jnp.float32),
        pltpu.VMEM_SHARED((N_PAD, D), jnp.float32),
        pltpu.SemaphoreType.DMA,
        pltpu.SemaphoreType.DMA,
        pltpu.SemaphoreType.DMA,
        pltpu.SemaphoreType.DMA,
        pltpu.SemaphoreType.DMA,
        pltpu.SemaphoreType.DMA,
        pltpu.SemaphoreType.DMA,
        pltpu.SemaphoreType.DMA,
        pltpu.SemaphoreType.DMA,
        pltpu.SemaphoreType.DMA,
        pltpu.SemaphoreType.DMA,
        pltpu.SemaphoreType.DMA,
        pltpu.SemaphoreType.DMA,
        pltpu.SemaphoreType.DMA,
        pltpu.SemaphoreType.DMA,
    ],
)
def _scatter_kernel(h_hbm, src_hbm, dst_hbm, norm_hbm, zeros_hbm, out_hbm,
                    r0, r1, r2, sv0, sv1, sv2, dv0, dv1, dv2, nv0, nv1, nv2,
                    acc_sh,
                    g0, g1, g2, s0, s1, s2,
                    is0, is1, is2, id0, id1, id2, in0, in1, in2):
    cid = lax.axis_index("c")
    sid = lax.axis_index("s")
    wid = cid * NS + sid
    rows = (r0, r1, r2)
    srcv = (sv0, sv1, sv2)
    dstv = (dv0, dv1, dv2)
    normv = (nv0, nv1, nv2)
    gsem = (g0, g1, g2)
    ssem = (s0, s1, s2)
    isem = (is0, is1, is2)
    idsem = (id0, id1, id2)
    insem = (in0, in1, in2)

    @pl.when(sid == 0)
    def _():
        pltpu.sync_copy(zeros_hbm, acc_sh)

    plsc.subcore_barrier()

    def chunk(i):
        return pl.ds((wid * CPW + i) * C, C)

    def istart_src(i, p):
        pltpu.async_copy(src_hbm.at[chunk(i)], srcv[p], isem[p])

    def iwait_src(p):
        pltpu.make_async_copy(src_hbm.at[chunk(0)], srcv[p], isem[p]).wait()

    def istart_dst(i, p):
        pltpu.async_copy(dst_hbm.at[chunk(i)], dstv[p], idsem[p])

    def iwait_dst(p):
        pltpu.make_async_copy(dst_hbm.at[chunk(0)], dstv[p], idsem[p]).wait()

    def istart_norm(i, p):
        pltpu.async_copy(norm_hbm.at[chunk(i)], normv[p], insem[p])

    def iwait_norm(p):
        pltpu.make_async_copy(norm_hbm.at[chunk(0)], normv[p], insem[p]).wait()

    def gstart(p):
        pltpu.async_copy(h_hbm.at[srcv[p]], rows[p], gsem[p])

    def gwait(p):
        pltpu.make_async_copy(h_hbm.at[srcv[p]], rows[p], gsem[p]).wait()

    def swait(p):
        pltpu.make_async_copy(rows[p], acc_sh.at[dstv[p]], ssem[p]).wait()

    def body(i, p, first, last):
        p2 = (p + 2) % 3
        if not last:
            istart_src(i + 2, p2)
            istart_norm(i + 2, p2)
        gwait(p)
        iwait_norm(p)

        @plsc.parallel_loop(0, C, unroll=4)
        def _(e):
            nv = plsc.load_gather(normv[p], [jnp.broadcast_to(e, (16,))])
            for k in range(D // 16):
                rows[p][e, pl.ds(k * 16, 16)] = rows[p][e, pl.ds(k * 16, 16)] * nv

        iwait_dst(p)
        pltpu.async_copy(rows[p], acc_sh.at[dstv[p]], ssem[p], add=True)
        if not first:
            swait(p2)
        if not last:
            istart_dst(i + 2, p2)
            iwait_src(p2)
            gstart(p2)

    for p in range(2):
        istart_src(p, p)
        istart_norm(p, p)
        istart_dst(p, p)
    iwait_src(0)
    gstart(0)
    iwait_src(1)
    gstart(1)
    body(0, 0, True, False)
    body(1, 1, False, False)

    @pl.loop(0, (CPW - 4) // 3)
    def _(k):
        i = 3 * k + 2
        body(i, 2, False, False)
        body(i + 1, 0, False, False)
        body(i + 2, 1, False, False)

    body(CPW - 2, 2, False, True)
    body(CPW - 1, 0, False, True)
    swait(0)

    plsc.subcore_barrier()

    @pl.when(sid == 0)
    def _():
        pltpu.sync_copy(acc_sh, out_hbm.at[cid])


def _tc_prep_body(parts_ref, x_ref, w_ref, dinv_ref, t_ref):
    deg = parts_ref[0, :N, 0:1] + parts_ref[1, :N, 0:1] + 1.0
    dinv = jnp.where(deg > 0.0, lax.rsqrt(jnp.abs(deg) + 1e-30), 0.0)
    dinv_ref[...] = dinv
    t_ref[...] = jnp.dot(x_ref[...], w_ref[...],
                         preferred_element_type=jnp.float32)


def _tc_prep(parts, x, w):
    return pl.pallas_call(
        _tc_prep_body,
        out_shape=[
            jax.ShapeDtypeStruct((N, 1), jnp.float32),
            jax.ShapeDtypeStruct((N, D), jnp.float32),
        ],
    )(parts, x, w)


def _tc_fuse_body(p_ref, t_ref, dinv_ref, b_ref, w_ref, out_ref):
    d2 = dinv_ref[...] * dinv_ref[...]
    agg = p_ref[0, :N] + p_ref[1, :N] + d2 * t_ref[...] + b_ref[...]
    h = jnp.maximum(agg, 0.0)
    out_ref[...] = jnp.dot(h, w_ref[...], preferred_element_type=jnp.float32)


def _tc_fuse(p, t, dinv, b, w):
    return pl.pallas_call(
        _tc_fuse_body,
        out_shape=jax.ShapeDtypeStruct((N, D), jnp.float32),
    )(p, t, dinv, b, w)


def _tc_final_body(p_ref, t_ref, dinv_ref, b_ref, batch_ref, wl_ref, bl_ref,
                   out_ref):
    d2 = dinv_ref[...] * dinv_ref[...]
    h3 = p_ref[0, :N] + p_ref[1, :N] + d2 * t_ref[...] + b_ref[...]
    gid = lax.broadcasted_iota(jnp.int32, (G, N), 0)
    onehot = (batch_ref[...] == gid).astype(jnp.float32)
    sums = jnp.dot(onehot, h3, preferred_element_type=jnp.float32)
    counts = jnp.sum(onehot, axis=1, keepdims=True)
    hg = sums / jnp.maximum(counts, 1.0)
    out_ref[...] = jnp.dot(hg, wl_ref[...],
                           preferred_element_type=jnp.float32) + bl_ref[...]


def _tc_final(p, t, dinv, b, batch2d, wl, bl):
    return pl.pallas_call(
        _tc_final_body,
        out_shape=jax.ShapeDtypeStruct((G, DO), jnp.float32),
    )(p, t, dinv, b, batch2d, wl, bl)


def kernel(x, edge_index, edge_weight, batch, W1, b1, W2, b2, W3, b3, Wl, bl):
    src = edge_index[0]
    dst = edge_index[1]
    npad = E_PAD - E
    pad_idx = (jnp.arange(npad, dtype=jnp.int32) % N)
    src_p = jnp.concatenate([src.astype(jnp.int32), pad_idx])
    dst_p = jnp.concatenate([dst.astype(jnp.int32), pad_idx])
    w_p = jnp.concatenate([edge_weight, jnp.zeros((npad,), jnp.float32)])

    zeros_nd = jnp.zeros((N_PAD, D), jnp.float32)
    parts = _deg_kernel(dst_p, w_p, zeros_nd)
    dinv2d, t1 = _tc_prep(parts, x, W1)
    norm = _norm_kernel(src_p, dst_p, w_p, dinv2d.reshape(N))

    p1 = _scatter_kernel(t1, src_p, dst_p, norm, zeros_nd)
    t2 = _tc_fuse(p1, t1, dinv2d, b1.reshape(1, D), W2)
    p2 = _scatter_kernel(t2, src_p, dst_p, norm, zeros_nd)
    t3 = _tc_fuse(p2, t2, dinv2d, b2.reshape(1, D), W3)
    p3 = _scatter_kernel(t3, src_p, dst_p, norm, zeros_nd)

    return _tc_final(p3, t3, dinv2d, b3.reshape(1, D),
                     batch.reshape(1, N).astype(jnp.int32),
                     Wl, bl.reshape(1, DO))
